# baseline (speedup 1.0000x reference)
import functools

import jax
import jax.numpy as jnp
from jax import lax
from jax.experimental import pallas as pl
from jax.experimental.pallas import tpu as pltpu
from jax.experimental.pallas import tpu_sc as plsc

N = 256
H = 256
A = 64
NW = 32
ROWS = N // NW
HALF = N // 2
L = 16
HC = H // L


def _prep(t_ref, ts_ref, ss_ref, w1_ref, b1_ref, w2_ref, b2_ref,
          v_ref, bias_ref):
    tp2 = jax.lax.dot_general(
        t_ref[...], w2_ref[...], (((1,), (1,)), ((), ())),
        preferred_element_type=jnp.float32) + b2_ref[0, :][None, :]
    m = jnp.logical_and(ts_ref[0, :] == 1.0,
                        ss_ref[0, :] == 0.0).astype(jnp.float32)
    en = jnp.sum(m)
    scale = en * jax.lax.rsqrt(jnp.float32(A))
    v_ref[...] = jax.lax.dot_general(
        tp2, w1_ref[...], (((1,), (0,)), ((), ())),
        preferred_element_type=jnp.float32) * scale
    c = jnp.sum(tp2 * b1_ref[0, :][None, :], axis=1,
                keepdims=True) * scale
    rows = jax.lax.broadcasted_iota(jnp.int32, (N, N), 0)
    cols = jax.lax.broadcasted_iota(jnp.int32, (N, N), 1)
    allowed = ((rows != cols) & (m[:, None] > 0.0) & (m[None, :] > 0.0))
    bias_ref[...] = c + jnp.where(allowed, 0.0, -1e30)


def _sc_body(s_hbm, v_hbm, bias_hbm, out_hbm,
             buf_a, buf_b, v_buf, bias_buf, out_buf, sem_a, sem_b):
    wid = lax.axis_index("s") * 2 + lax.axis_index("c")

    iota = lax.iota(jnp.int32, L)
    rots = [jnp.bitwise_and(iota + sh, L - 1) for sh in (8, 4, 2, 1)]

    gdn = lax.GatherDimensionNumbers(
        offset_dims=(), collapsed_slice_dims=(0,), start_index_map=(0,))

    def perm(x, idx):
        return lax.gather(x, idx[:, None], gdn, (1,),
                          mode=lax.GatherScatterMode.PROMISE_IN_BOUNDS)

    def splat_sum(x):
        for rot in rots:
            x = x + perm(x, rot)
        return x

    def compute_half(buf, j_base, carry):
        vs = [v_buf[pl.ds(k * L, L)] for k in range(HC)]

        def body(jl, carry):
            den = carry[0]
            accs = carry[1:]
            j = j_base + jl
            xs = [buf[jl, pl.ds(k * L, L)] for k in range(HC)]
            prods = [xs[k] * vs[k] for k in range(HC)]
            while len(prods) > 1:
                prods = [prods[2 * k] + prods[2 * k + 1]
                         for k in range(len(prods) // 2)]
            t = splat_sum(prods[0])
            b_j = plsc.load_gather(bias_buf, [jnp.broadcast_to(j, (L,))])
            e = jnp.exp(t + b_j)
            new_accs = [accs[k] + e * xs[k] for k in range(HC)]
            return (den + e, *new_accs)

        return lax.fori_loop(0, HALF, body, carry)

    for r in range(ROWS):
        row = wid * ROWS + r
        pltpu.sync_copy(v_hbm.at[row], v_buf)
        pltpu.sync_copy(bias_hbm.at[row], bias_buf)
        cp_a = pltpu.async_copy(s_hbm.at[0, row, pl.ds(0, HALF)],
                                buf_a, sem_a)
        cp_b = pltpu.async_copy(s_hbm.at[0, row, pl.ds(HALF, HALF)],
                                buf_b, sem_b)
        zero = jnp.zeros((L,), jnp.float32)
        carry = (zero,) * (HC + 1)
        cp_a.wait()
        carry = compute_half(buf_a, 0, carry)
        cp_b.wait()
        carry = compute_half(buf_b, HALF, carry)
        den = carry[0]
        inv = 1.0 / jnp.where(den == 0.0, 1.0, den)
        for k in range(HC):
            out_buf[pl.ds(k * L, L)] = carry[1 + k] * inv
        pltpu.sync_copy(out_buf, out_hbm.at[row])


@jax.jit
def _edge_to_node_attention(spatial_ht_list, temporal_ht_list, ts_mask,
                            same_scene_mask, W1_w, W1_b, W2_w, W2_b):
    v, bias = pl.pallas_call(
        _prep,
        out_shape=(
            jax.ShapeDtypeStruct((N, H), jnp.float32),
            jax.ShapeDtypeStruct((N, N), jnp.float32),
        ),
    )(temporal_ht_list, ts_mask, same_scene_mask, W1_w, W1_b, W2_w, W2_b)

    mesh = plsc.VectorSubcoreMesh(core_axis_name="c", subcore_axis_name="s")
    sc = pl.kernel(
        _sc_body,
        out_type=jax.ShapeDtypeStruct((N, H), jnp.float32),
        mesh=mesh,
        scratch_types=[
            pltpu.VMEM((HALF, H), jnp.float32),
            pltpu.VMEM((HALF, H), jnp.float32),
            pltpu.VMEM((H,), jnp.float32),
            pltpu.VMEM((N,), jnp.float32),
            pltpu.VMEM((H,), jnp.float32),
            pltpu.SemaphoreType.DMA,
            pltpu.SemaphoreType.DMA,
        ],
        compiler_params=pltpu.CompilerParams(needs_layout_passes=False),
    )
    return sc(spatial_ht_list, v, bias)


def kernel(spatial_ht_list, temporal_ht_list, ts_mask, same_scene_mask,
           W1_w, W1_b, W2_w, W2_b):
    return _edge_to_node_attention(
        spatial_ht_list, temporal_ht_list,
        ts_mask.reshape(1, N), same_scene_mask.reshape(1, N),
        W1_w, W1_b.reshape(1, A), W2_w, W2_b.reshape(1, A))


# speedup vs baseline: 1.2450x; 1.2450x over previous
import functools

import jax
import jax.numpy as jnp
from jax import lax
from jax.experimental import pallas as pl
from jax.experimental.pallas import tpu as pltpu
from jax.experimental.pallas import tpu_sc as plsc

N = 256
H = 256
A = 64
NW = 32
ROWS = N // NW
HALF = N // 2
L = 16
HC = H // L


def _prep(t_ref, ts_ref, ss_ref, w1_ref, b1_ref, w2_ref, b2_ref,
          v_ref, bias_ref):
    tp2 = jax.lax.dot_general(
        t_ref[...], w2_ref[...], (((1,), (1,)), ((), ())),
        preferred_element_type=jnp.float32) + b2_ref[0, :][None, :]
    m = jnp.logical_and(ts_ref[0, :] == 1.0,
                        ss_ref[0, :] == 0.0).astype(jnp.float32)
    en = jnp.sum(m)
    scale = en * jax.lax.rsqrt(jnp.float32(A))
    v_ref[...] = jax.lax.dot_general(
        tp2, w1_ref[...], (((1,), (0,)), ((), ())),
        preferred_element_type=jnp.float32) * scale
    c = jnp.sum(tp2 * b1_ref[0, :][None, :], axis=1,
                keepdims=True) * scale
    rows = jax.lax.broadcasted_iota(jnp.int32, (N, N), 0)
    cols = jax.lax.broadcasted_iota(jnp.int32, (N, N), 1)
    allowed = ((rows != cols) & (m[:, None] > 0.0) & (m[None, :] > 0.0))
    bias_ref[...] = c + jnp.where(allowed, 0.0, -1e30)


def _sc_body(s_hbm, v_hbm, bias_hbm, out_hbm,
             buf_a, buf_b, v_buf, bias_buf, out_buf, sem_a, sem_b):
    wid = lax.axis_index("s") * 2 + lax.axis_index("c")

    iota = lax.iota(jnp.int32, L)
    rots = [jnp.bitwise_and(iota + sh, L - 1) for sh in (8, 4, 2, 1)]

    gdn = lax.GatherDimensionNumbers(
        offset_dims=(), collapsed_slice_dims=(0,), start_index_map=(0,))

    def perm(x, idx):
        return lax.gather(x, idx[:, None], gdn, (1,),
                          mode=lax.GatherScatterMode.PROMISE_IN_BOUNDS)

    def splat_sum(x):
        for rot in rots:
            x = x + perm(x, rot)
        return x

    U = 4

    def compute_half(buf, j_base, carry):
        vs = [v_buf[pl.ds(k * L, L)] for k in range(HC)]

        def one_edge(jl):
            xs = [buf[jl, pl.ds(k * L, L)] for k in range(HC)]
            prods = [xs[k] * vs[k] for k in range(HC)]
            while len(prods) > 1:
                prods = [prods[2 * k] + prods[2 * k + 1]
                         for k in range(len(prods) // 2)]
            t = splat_sum(prods[0])
            b_j = plsc.load_gather(
                bias_buf, [jnp.broadcast_to(j_base + jl, (L,))])
            e = jnp.exp(t + b_j)
            return e, xs

        def body(g, carry):
            den = carry[0]
            accs = list(carry[1:])
            exs = [one_edge(g * U + u) for u in range(U)]
            for e, xs in exs:
                den = den + e
                accs = [accs[k] + e * xs[k] for k in range(HC)]
            return (den, *accs)

        return lax.fori_loop(0, HALF // U, body, carry)

    for r in range(ROWS):
        row = wid * ROWS + r
        pltpu.sync_copy(v_hbm.at[row], v_buf)
        pltpu.sync_copy(bias_hbm.at[row], bias_buf)
        cp_a = pltpu.async_copy(s_hbm.at[0, row, pl.ds(0, HALF)],
                                buf_a, sem_a)
        cp_b = pltpu.async_copy(s_hbm.at[0, row, pl.ds(HALF, HALF)],
                                buf_b, sem_b)
        zero = jnp.zeros((L,), jnp.float32)
        carry = (zero,) * (HC + 1)
        cp_a.wait()
        carry = compute_half(buf_a, 0, carry)
        cp_b.wait()
        carry = compute_half(buf_b, HALF, carry)
        den = carry[0]
        inv = 1.0 / jnp.where(den == 0.0, 1.0, den)
        for k in range(HC):
            out_buf[pl.ds(k * L, L)] = carry[1 + k] * inv
        pltpu.sync_copy(out_buf, out_hbm.at[row])


@jax.jit
def _edge_to_node_attention(spatial_ht_list, temporal_ht_list, ts_mask,
                            same_scene_mask, W1_w, W1_b, W2_w, W2_b):
    v, bias = pl.pallas_call(
        _prep,
        out_shape=(
            jax.ShapeDtypeStruct((N, H), jnp.float32),
            jax.ShapeDtypeStruct((N, N), jnp.float32),
        ),
    )(temporal_ht_list, ts_mask, same_scene_mask, W1_w, W1_b, W2_w, W2_b)

    mesh = plsc.VectorSubcoreMesh(core_axis_name="c", subcore_axis_name="s")
    sc = pl.kernel(
        _sc_body,
        out_type=jax.ShapeDtypeStruct((N, H), jnp.float32),
        mesh=mesh,
        scratch_types=[
            pltpu.VMEM((HALF, H), jnp.float32),
            pltpu.VMEM((HALF, H), jnp.float32),
            pltpu.VMEM((H,), jnp.float32),
            pltpu.VMEM((N,), jnp.float32),
            pltpu.VMEM((H,), jnp.float32),
            pltpu.SemaphoreType.DMA,
            pltpu.SemaphoreType.DMA,
        ],
        compiler_params=pltpu.CompilerParams(needs_layout_passes=False),
    )
    return sc(spatial_ht_list, v, bias)


def kernel(spatial_ht_list, temporal_ht_list, ts_mask, same_scene_mask,
           W1_w, W1_b, W2_w, W2_b):
    return _edge_to_node_attention(
        spatial_ht_list, temporal_ht_list,
        ts_mask.reshape(1, N), same_scene_mask.reshape(1, N),
        W1_w, W1_b.reshape(1, A), W2_w, W2_b.reshape(1, A))


# speedup vs baseline: 2.9756x; 2.3901x over previous
import jax
import jax.numpy as jnp
from jax import lax
from jax.experimental import pallas as pl
from jax.experimental.pallas import tpu as pltpu
from jax.experimental.pallas import tpu_sc as plsc

N = 256
H = 256
A = 64
SPLIT = 192
BI = 16
NW = 32
ROWS = (N - SPLIT) // NW
HALF = N // 2
L = 16
HC = H // L


def _prep(t_ref, ts_ref, ss_ref, w1_ref, b1_ref, w2_ref, b2_ref,
          v_ref, bias_ref):
    tp2 = jax.lax.dot_general(
        t_ref[...], w2_ref[...], (((1,), (1,)), ((), ())),
        preferred_element_type=jnp.float32) + b2_ref[0, :][None, :]
    m = jnp.logical_and(ts_ref[0, :] == 1.0,
                        ss_ref[0, :] == 0.0).astype(jnp.float32)
    en = jnp.sum(m)
    scale = en * jax.lax.rsqrt(jnp.float32(A))
    v_ref[...] = jax.lax.dot_general(
        tp2, w1_ref[...], (((1,), (0,)), ((), ())),
        preferred_element_type=jnp.float32) * scale
    c = jnp.sum(tp2 * b1_ref[0, :][None, :], axis=1,
                keepdims=True) * scale
    rows = jax.lax.broadcasted_iota(jnp.int32, (N, N), 0)
    cols = jax.lax.broadcasted_iota(jnp.int32, (N, N), 1)
    allowed = ((rows != cols) & (m[:, None] > 0.0) & (m[None, :] > 0.0))
    bias_ref[...] = c + jnp.where(allowed, 0.0, -1e30)


def _attn_block(s_ref, v_ref, bias_ref, out_ref):
    s3 = s_ref[0]
    s2 = s3.reshape(BI * N, H)
    p = jax.lax.dot_general(
        s2, v_ref[...], (((1,), (1,)), ((), ())),
        preferred_element_type=jnp.float32)
    p3 = p.reshape(BI, N, BI)
    eye = (jax.lax.broadcasted_iota(jnp.int32, (BI, 1, BI), 0) ==
           jax.lax.broadcasted_iota(jnp.int32, (BI, 1, BI), 2)
           ).astype(jnp.float32)
    sm = jnp.sum(p3 * eye, axis=2)
    num = jnp.exp(sm + bias_ref[...])
    den = jnp.sum(num, axis=1, keepdims=True)
    inv = 1.0 / jnp.where(den == 0.0, 1.0, den)
    score = num * inv
    out_ref[...] = jnp.sum(s3 * score[:, :, None], axis=1)


def _sc_body(s_hbm, v_hbm, bias_hbm, out_hbm,
             buf_a, buf_b, v_buf, bias_buf, out_buf, sem_a, sem_b):
    wid = lax.axis_index("s") * 2 + lax.axis_index("c")

    iota = lax.iota(jnp.int32, L)
    rots = [jnp.bitwise_and(iota + sh, L - 1) for sh in (8, 4, 2, 1)]
    gdn = lax.GatherDimensionNumbers(
        offset_dims=(), collapsed_slice_dims=(0,), start_index_map=(0,))

    def perm(x, idx):
        return lax.gather(x, idx[:, None], gdn, (1,),
                          mode=lax.GatherScatterMode.PROMISE_IN_BOUNDS)

    def splat_sum(x):
        for rot in rots:
            x = x + perm(x, rot)
        return x

    U = 4

    def compute_half(buf, j_base, carry):
        vs = [v_buf[pl.ds(k * L, L)] for k in range(HC)]

        def one_edge(jl):
            xs = [buf[jl, pl.ds(k * L, L)] for k in range(HC)]
            prods = [xs[k] * vs[k] for k in range(HC)]
            while len(prods) > 1:
                prods = [prods[2 * k] + prods[2 * k + 1]
                         for k in range(len(prods) // 2)]
            t = splat_sum(prods[0])
            b_j = plsc.load_gather(
                bias_buf, [jnp.broadcast_to(j_base + jl, (L,))])
            e = jnp.exp(t + b_j)
            return e, xs

        def body(g, carry):
            den = carry[0]
            accs = list(carry[1:])
            exs = [one_edge(g * U + u) for u in range(U)]
            for e, xs in exs:
                den = den + e
                accs = [accs[k] + e * xs[k] for k in range(HC)]
            return (den, *accs)

        return lax.fori_loop(0, HALF // U, body, carry)

    for r in range(ROWS):
        row_local = wid * ROWS + r
        row = SPLIT + row_local
        pltpu.sync_copy(v_hbm.at[row], v_buf)
        pltpu.sync_copy(bias_hbm.at[row], bias_buf)
        cp_a = pltpu.async_copy(s_hbm.at[0, row, pl.ds(0, HALF)],
                                buf_a, sem_a)
        cp_b = pltpu.async_copy(s_hbm.at[0, row, pl.ds(HALF, HALF)],
                                buf_b, sem_b)
        zero = jnp.zeros((L,), jnp.float32)
        carry = (zero,) * (HC + 1)
        cp_a.wait()
        carry = compute_half(buf_a, 0, carry)
        cp_b.wait()
        carry = compute_half(buf_b, HALF, carry)
        den = carry[0]
        inv = 1.0 / jnp.where(den == 0.0, 1.0, den)
        for k in range(HC):
            out_buf[pl.ds(k * L, L)] = carry[1 + k] * inv
        pltpu.sync_copy(out_buf, out_hbm.at[row_local])


@jax.jit
def _edge_to_node_attention(spatial_ht_list, temporal_ht_list, ts_mask,
                            same_scene_mask, W1_w, W1_b, W2_w, W2_b):
    v, bias = pl.pallas_call(
        _prep,
        out_shape=(
            jax.ShapeDtypeStruct((N, H), jnp.float32),
            jax.ShapeDtypeStruct((N, N), jnp.float32),
        ),
    )(temporal_ht_list, ts_mask, same_scene_mask, W1_w, W1_b, W2_w, W2_b)

    mesh = plsc.VectorSubcoreMesh(core_axis_name="c", subcore_axis_name="s")
    sc = pl.kernel(
        _sc_body,
        out_type=jax.ShapeDtypeStruct((N - SPLIT, H), jnp.float32),
        mesh=mesh,
        scratch_types=[
            pltpu.VMEM((HALF, H), jnp.float32),
            pltpu.VMEM((HALF, H), jnp.float32),
            pltpu.VMEM((H,), jnp.float32),
            pltpu.VMEM((N,), jnp.float32),
            pltpu.VMEM((H,), jnp.float32),
            pltpu.SemaphoreType.DMA,
            pltpu.SemaphoreType.DMA,
        ],
        compiler_params=pltpu.CompilerParams(needs_layout_passes=False),
    )
    out_sc = sc(spatial_ht_list, v, bias)

    out_tc = pl.pallas_call(
        _attn_block,
        grid=(SPLIT // BI,),
        in_specs=[
            pl.BlockSpec((1, BI, N, H), lambda i: (0, i, 0, 0)),
            pl.BlockSpec((BI, H), lambda i: (i, 0)),
            pl.BlockSpec((BI, N), lambda i: (i, 0)),
        ],
        out_specs=pl.BlockSpec((BI, H), lambda i: (i, 0)),
        out_shape=jax.ShapeDtypeStruct((SPLIT, H), jnp.float32),
    )(spatial_ht_list, v, bias)

    return jnp.concatenate([out_tc, out_sc], axis=0)


def kernel(spatial_ht_list, temporal_ht_list, ts_mask, same_scene_mask,
           W1_w, W1_b, W2_w, W2_b):
    return _edge_to_node_attention(
        spatial_ht_list, temporal_ht_list,
        ts_mask.reshape(1, N), same_scene_mask.reshape(1, N),
        W1_w, W1_b.reshape(1, A), W2_w, W2_b.reshape(1, A))
